# Initial kernel scaffold; baseline (speedup 1.0000x reference)
import jax
import jax.numpy as jnp
from jax.experimental import pallas as pl


def kernel(x, gate_W, gate_b, w1_W, w1_b, w2_W, w2_b, w3_W, w3_b):
    raise NotImplementedError("write your pallas kernel here")


# speedup vs baseline: 1.0783x; 1.0783x over previous
import functools

import jax
import jax.numpy as jnp
from jax import lax
from jax.experimental import pallas as pl
from jax.experimental.pallas import tpu as pltpu

NEG_BIG = -1e30


def _router_body(x_ref, gw_ref, gb_ref, w_ref, loss_ref):
    x = x_ref[:]
    gw = gw_ref[:]
    logits = lax.dot_general(x, gw, (((1,), (1,)), ((), ())),
                             preferred_element_type=jnp.float32)
    logits = logits + gb_ref[:]
    T, EP = logits.shape
    m = jnp.max(logits, axis=1, keepdims=True)
    p = jnp.exp(logits - m)
    probs = p / jnp.sum(p, axis=1, keepdims=True)

    lane = lax.broadcasted_iota(jnp.int32, (T, EP), 1)
    m1 = jnp.max(probs, axis=1, keepdims=True)
    a1 = jnp.min(jnp.where(probs == m1, lane, EP), axis=1, keepdims=True)
    probs_wo1 = jnp.where(lane == a1, -1.0, probs)
    m2 = jnp.max(probs_wo1, axis=1, keepdims=True)
    a2 = jnp.min(jnp.where(probs_wo1 == m2, lane, EP), axis=1, keepdims=True)
    w = jnp.where(lane == a1, m1, 0.0) + jnp.where(lane == a2, m2, 0.0)
    w_ref[:] = w[:, :w_ref.shape[1]]

    usage = jnp.sum(probs, axis=0, keepdims=True) / T
    ul = usage * jnp.log(usage + 1e-9)
    ul = jnp.where(lax.broadcasted_iota(jnp.int32, (1, EP), 1) < w_ref.shape[1],
                   ul, 0.0)
    loss_ref[0, 0] = jnp.sum(ul)


def _expert_body(x_ref, w1_ref, b1_ref, w2_ref, b2_ref, w3_ref, b3_ref,
                 wc_ref, out_ref, acc_ref):
    e = pl.program_id(1)
    j = pl.program_id(2)
    nj = pl.num_programs(2)

    x = x_ref[:]
    w1 = w1_ref[0]
    w2 = w2_ref[0]
    h = lax.dot_general(x, w1, (((1,), (1,)), ((), ())),
                        preferred_element_type=jnp.float32) + b1_ref[0]
    g = lax.dot_general(x, w2, (((1,), (1,)), ((), ())),
                        preferred_element_type=jnp.float32) + b2_ref[0]
    hg = h * jax.nn.sigmoid(g)
    w3 = w3_ref[0]
    part = lax.dot_general(hg, w3, (((1,), (1,)), ((), ())),
                           preferred_element_type=jnp.float32)

    @pl.when(j == 0)
    def _():
        acc_ref[:] = part

    @pl.when(j > 0)
    def _():
        acc_ref[:] = acc_ref[:] + part

    @pl.when(jnp.logical_and(e == 0, j == 0))
    def _():
        out_ref[:] = jnp.zeros_like(out_ref)

    @pl.when(j == nj - 1)
    def _():
        wc = wc_ref[:]
        lane = lax.broadcasted_iota(jnp.int32, wc.shape, 1)
        col = jnp.sum(jnp.where(lane == e, wc, 0.0), axis=1, keepdims=True)
        out_ref[:] = out_ref[:] + col * (acc_ref[:] + b3_ref[0])


def kernel(x, gate_W, gate_b, w1_W, w1_b, w2_W, w2_b, w3_W, w3_b):
    B, S, DIM = x.shape
    E, HIDDEN = w1_b.shape
    T = B * S
    xf = x.reshape(T, DIM)

    EP = 128
    gw_p = jnp.zeros((EP, DIM), jnp.float32).at[:E].set(gate_W)
    gb_p = jnp.full((1, EP), NEG_BIG, jnp.float32).at[0, :E].set(gate_b)

    wc, loss2 = pl.pallas_call(
        _router_body,
        out_shape=(
            jax.ShapeDtypeStruct((T, E), jnp.float32),
            jax.ShapeDtypeStruct((1, 1), jnp.float32),
        ),
        in_specs=[
            pl.BlockSpec((T, DIM), lambda: (0, 0)),
            pl.BlockSpec((EP, DIM), lambda: (0, 0)),
            pl.BlockSpec((1, EP), lambda: (0, 0)),
        ],
        out_specs=(
            pl.BlockSpec((T, E), lambda: (0, 0)),
            pl.BlockSpec(memory_space=pltpu.SMEM),
        ),
    )(xf, gw_p, gb_p)

    HB = min(256, HIDDEN)
    TM = min(2048, T)
    nj = HIDDEN // HB
    nm = T // TM
    grid = (nm, E, nj)
    out = pl.pallas_call(
        _expert_body,
        grid=grid,
        out_shape=jax.ShapeDtypeStruct((T, DIM), jnp.float32),
        in_specs=[
            pl.BlockSpec((TM, DIM), lambda m, e, j: (m, 0)),
            pl.BlockSpec((1, HB, DIM), lambda m, e, j: (e, j, 0)),
            pl.BlockSpec((1, 1, HB), lambda m, e, j: (e, 0, j)),
            pl.BlockSpec((1, HB, DIM), lambda m, e, j: (e, j, 0)),
            pl.BlockSpec((1, 1, HB), lambda m, e, j: (e, 0, j)),
            pl.BlockSpec((1, DIM, HB), lambda m, e, j: (e, 0, j)),
            pl.BlockSpec((1, 1, DIM), lambda m, e, j: (e, 0, 0)),
            pl.BlockSpec((TM, E), lambda m, e, j: (m, 0)),
        ],
        out_specs=pl.BlockSpec((TM, DIM), lambda m, e, j: (m, 0)),
        scratch_shapes=[pltpu.VMEM((TM, DIM), jnp.float32)],
    )(xf, w1_W, w1_b.reshape(E, 1, HIDDEN), w2_W, w2_b.reshape(E, 1, HIDDEN),
      w3_W, w3_b.reshape(E, 1, DIM), wc)

    return out.reshape(B, S, DIM), loss2[0, 0]
